# Initial kernel scaffold; baseline (speedup 1.0000x reference)
import jax
import jax.numpy as jnp
from jax.experimental import pallas as pl


def kernel(x, edge_index, deg_hist, W_pre1, b_pre1, W_post1, b_post1, W_lin1, b_lin1, W_pre2, b_pre2, W_post2, b_post2, W_lin2, b_lin2):
    raise NotImplementedError("write your pallas kernel here")


# speedup vs baseline: 91.0709x; 91.0709x over previous
import functools

import jax
import jax.numpy as jnp
from jax import lax
from jax.experimental import pallas as pl
from jax.experimental.pallas import tpu as pltpu
from jax.experimental.pallas import tpu_sc as plsc

N = 10000
E = 320000
D = 128
NPAD = 10240
NC = 2
NS = 16
NW = NC * NS
NPT = NPAD // NW
CAP = 11264
K = 32
CH = 1600
BLK = 1024


def _sc_mesh():
    return plsc.VectorSubcoreMesh(
        core_axis_name="c", subcore_axis_name="s", num_cores=NC, num_subcores=NS)


def _filter_body(src_hbm, dst_hbm, cnt_hbm, srcl_hbm, dstl_hbm,
                 sbuf, dbuf, slist, dlist, cvec):
    wid = lax.axis_index("s") * NC + lax.axis_index("c")
    lo = wid * NPT
    hi = lo + NPT

    def fill(i, pos):
        slist[pl.ds(pos, 16)] = jnp.zeros((16,), jnp.int32) + wid
        dlist[pl.ds(pos, 16)] = jnp.zeros((16,), jnp.int32) + NPT
        return pos + 16
    lax.fori_loop(0, CAP // 16, fill, jnp.int32(0))

    def chunk(c, carry):
        off, ebase = carry
        pltpu.sync_copy(src_hbm.at[pl.ds(pl.multiple_of(ebase, 8), CH)], sbuf)
        pltpu.sync_copy(dst_hbm.at[pl.ds(pl.multiple_of(ebase, 8), CH)], dbuf)

        def step(j, carry2):
            off, jpos = carry2
            dv = dbuf[pl.ds(jpos, 16)]
            sv = sbuf[pl.ds(jpos, 16)]
            m = jnp.logical_and(dv >= lo, dv < hi)
            m = jnp.logical_and(m, off <= CAP - 16)
            cs = plsc.cumsum(m.astype(jnp.int32))
            dest = off + cs - 1
            plsc.store_scatter(slist, [dest], sv, mask=m)
            plsc.store_scatter(dlist, [dest], dv - lo, mask=m)
            off = off + cs[15]
            return (off, jpos + 16)
        off, _ = lax.fori_loop(0, CH // 16, step, (off, jnp.int32(0)))
        return (off, ebase + CH)

    off, _ = lax.fori_loop(0, E // CH, chunk, (jnp.int32(0), jnp.int32(0)))
    cvec[...] = jnp.zeros((16,), jnp.int32) + off
    pltpu.sync_copy(cvec, cnt_hbm.at[pl.ds(wid * 16, 16)])
    pltpu.sync_copy(slist, srcl_hbm.at[pl.ds(wid * CAP, CAP)])
    pltpu.sync_copy(dlist, dstl_hbm.at[pl.ds(wid * CAP, CAP)])


_filter = pl.kernel(
    _filter_body,
    out_type=[
        jax.ShapeDtypeStruct((NW * 16,), jnp.int32),
        jax.ShapeDtypeStruct((NW * CAP,), jnp.int32),
        jax.ShapeDtypeStruct((NW * CAP,), jnp.int32),
    ],
    mesh=_sc_mesh(),
    scratch_types=[
        pltpu.VMEM((CH,), jnp.int32),
        pltpu.VMEM((CH,), jnp.int32),
        pltpu.VMEM((CAP,), jnp.int32),
        pltpu.VMEM((CAP,), jnp.int32),
        pltpu.VMEM((16,), jnp.int32),
    ],
    compiler_params=pltpu.CompilerParams(needs_layout_passes=False),
    name="pna_sc_filter",
)


def _agg_body(b_hbm, cnt_hbm, srcl_hbm, dstl_hbm,
              s1_hbm, s2_hbm, mn_hbm, mx_hbm, deg_hbm,
              sbuf, dbuf, rows, acc_a, acc_b, accd, cvec, sem):
    wid = lax.axis_index("s") * NC + lax.axis_index("c")
    base = wid * NPT
    lbase = wid * CAP
    pltpu.sync_copy(cnt_hbm.at[pl.ds(wid * 16, 16)], cvec)
    cnt = cvec[pl.ds(0, 16)][0]
    nb = lax.div(cnt + jnp.int32(K - 1), jnp.int32(K))

    zeros = jnp.zeros((16,), jnp.float32)
    pinf = jnp.full((16,), jnp.inf, jnp.float32)
    ninf = jnp.full((16,), -jnp.inf, jnp.float32)
    ones = jnp.ones((16,), jnp.float32)

    for phase in (0, 1):
        def init(i, pos):
            for r in range(8):
                sl = pl.ds(r * 16, 16)
                acc_a[pos, sl] = zeros if phase == 0 else pinf
                acc_b[pos, sl] = zeros if phase == 0 else ninf
            if phase == 0:
                accd[pos] = zeros
            return pos + 1
        lax.fori_loop(0, NPT + 1, init, jnp.int32(0))

        def batch(b, boff):
            pltpu.sync_copy(srcl_hbm.at[pl.ds(pl.multiple_of(lbase + boff, 8), K)], sbuf)
            pltpu.sync_copy(dstl_hbm.at[pl.ds(pl.multiple_of(lbase + boff, 8), K)], dbuf)
            pltpu.async_copy(b_hbm.at[sbuf], rows, sem).wait()

            def grp(g, gpos):
                dvec = dbuf[pl.ds(gpos, 16)]
                for lane in range(16):
                    dloc = dvec[lane]
                    e = gpos + lane
                    for r in range(8):
                        sl = pl.ds(r * 16, 16)
                        v = rows[e, sl]
                        if phase == 0:
                            plsc.addupdate(acc_a.at[dloc, sl], v)
                            plsc.addupdate(acc_b.at[dloc, sl], v * v)
                        else:
                            acc_a[dloc, sl] = jnp.minimum(acc_a[dloc, sl], v)
                            acc_b[dloc, sl] = jnp.maximum(acc_b[dloc, sl], v)
                    if phase == 0:
                        plsc.addupdate(accd.at[dloc], ones)
                return gpos + 16
            lax.fori_loop(0, K // 16, grp, jnp.int32(0))
            return boff + K
        lax.fori_loop(0, nb, batch, jnp.int32(0))

        npt0 = pl.ds(jnp.int32(0), NPT)
        if phase == 0:
            pltpu.sync_copy(acc_a.at[npt0], s1_hbm.at[pl.ds(base, NPT)])
            pltpu.sync_copy(acc_b.at[npt0], s2_hbm.at[pl.ds(base, NPT)])
            pltpu.sync_copy(accd.at[npt0], deg_hbm.at[pl.ds(base, NPT)])
        else:
            pltpu.sync_copy(acc_a.at[npt0], mn_hbm.at[pl.ds(base, NPT)])
            pltpu.sync_copy(acc_b.at[npt0], mx_hbm.at[pl.ds(base, NPT)])


_agg = pl.kernel(
    _agg_body,
    out_type=[
        jax.ShapeDtypeStruct((NPAD, D), jnp.float32),
        jax.ShapeDtypeStruct((NPAD, D), jnp.float32),
        jax.ShapeDtypeStruct((NPAD, D), jnp.float32),
        jax.ShapeDtypeStruct((NPAD, D), jnp.float32),
        jax.ShapeDtypeStruct((NPAD, 16), jnp.float32),
    ],
    mesh=_sc_mesh(),
    scratch_types=[
        pltpu.VMEM((K,), jnp.int32),
        pltpu.VMEM((K,), jnp.int32),
        pltpu.VMEM((K, D), jnp.float32),
        pltpu.VMEM((NPT + 1, D), jnp.float32),
        pltpu.VMEM((NPT + 1, D), jnp.float32),
        pltpu.VMEM((NPT + 1, 16), jnp.float32),
        pltpu.VMEM((16,), jnp.int32),
        pltpu.SemaphoreType.DMA,
    ],
    compiler_params=pltpu.CompilerParams(needs_layout_passes=False),
    name="pna_sc_agg",
)


def _pre_body(x_ref, wd_ref, ws_ref, bp_ref, a_ref, b_ref):
    xb = x_ref[...]
    a_ref[...] = (jnp.dot(xb, wd_ref[...], preferred_element_type=jnp.float32, precision=lax.Precision.HIGHEST)
                  + bp_ref[...])
    b_ref[...] = jnp.dot(xb, ws_ref[...], preferred_element_type=jnp.float32, precision=lax.Precision.HIGHEST)


_pre = pl.pallas_call(
    _pre_body,
    grid=(NPAD // BLK,),
    in_specs=[
        pl.BlockSpec((BLK, D), lambda i: (i, jnp.int32(0))),
        pl.BlockSpec((D, D), lambda i: (jnp.int32(0), jnp.int32(0))),
        pl.BlockSpec((D, D), lambda i: (jnp.int32(0), jnp.int32(0))),
        pl.BlockSpec((1, D), lambda i: (jnp.int32(0), jnp.int32(0))),
    ],
    out_specs=[
        pl.BlockSpec((BLK, D), lambda i: (i, jnp.int32(0))),
        pl.BlockSpec((BLK, D), lambda i: (i, jnp.int32(0))),
    ],
    out_shape=[
        jax.ShapeDtypeStruct((NPAD, D), jnp.float32),
        jax.ShapeDtypeStruct((NPAD, D), jnp.float32),
    ],
)


def _post_body(x_ref, a_ref, s1_ref, s2_ref, mn_ref, mx_ref, deg_ref, avg_ref,
               wp_ref, bp_ref, wl_ref, bl_ref, o_ref, *, relu):
    xb = x_ref[...]
    A = a_ref[...]
    S1 = s1_ref[...]
    S2 = s2_ref[...]
    MN = mn_ref[...]
    MX = mx_ref[...]
    deg = deg_ref[...][:, :1]
    degc = jnp.maximum(deg, 1.0)
    has = deg > 0
    logd = jnp.log(degc + 1.0)
    mean = (deg * A + S1) / degc
    meanB = S1 / degc
    varB = S2 / degc - meanB * meanB
    std = jnp.sqrt(jnp.maximum(varB, 0.0) + 1e-5)
    mn = jnp.where(has, A + MN, 0.0)
    mx = jnp.where(has, A + MX, 0.0)
    avg = avg_ref[...][:1, :1]
    sa = logd / avg
    sb = avg / logd
    parts = (xb, mean, mn, mx, std,
             sa * mean, sa * mn, sa * mx, sa * std,
             sb * mean, sb * mn, sb * mx, sb * std)
    y = bp_ref[...] + jnp.zeros((BLK, D), jnp.float32)
    for j in range(13):
        y = y + jnp.dot(parts[j], wp_ref[j], preferred_element_type=jnp.float32, precision=lax.Precision.HIGHEST)
    z = jnp.dot(y, wl_ref[...], preferred_element_type=jnp.float32, precision=lax.Precision.HIGHEST) + bl_ref[...]
    o_ref[...] = jnp.maximum(z, 0.0) if relu else z


def _make_post(relu):
    return pl.pallas_call(
        functools.partial(_post_body, relu=relu),
        grid=(NPAD // BLK,),
        in_specs=[
            pl.BlockSpec((BLK, D), lambda i: (i, jnp.int32(0))),
            pl.BlockSpec((BLK, D), lambda i: (i, jnp.int32(0))),
            pl.BlockSpec((BLK, D), lambda i: (i, jnp.int32(0))),
            pl.BlockSpec((BLK, D), lambda i: (i, jnp.int32(0))),
            pl.BlockSpec((BLK, D), lambda i: (i, jnp.int32(0))),
            pl.BlockSpec((BLK, D), lambda i: (i, jnp.int32(0))),
            pl.BlockSpec((BLK, 16), lambda i: (i, jnp.int32(0))),
            pl.BlockSpec((8, 128), lambda i: (jnp.int32(0), jnp.int32(0))),
            pl.BlockSpec((13, D, D), lambda i: (jnp.int32(0), jnp.int32(0), jnp.int32(0))),
            pl.BlockSpec((1, D), lambda i: (jnp.int32(0), jnp.int32(0))),
            pl.BlockSpec((D, D), lambda i: (jnp.int32(0), jnp.int32(0))),
            pl.BlockSpec((1, D), lambda i: (jnp.int32(0), jnp.int32(0))),
        ],
        out_specs=pl.BlockSpec((BLK, D), lambda i: (i, jnp.int32(0))),
        out_shape=jax.ShapeDtypeStruct((NPAD, D), jnp.float32),
    )


_post_relu = _make_post(True)
_post_none = _make_post(False)


def kernel(x, edge_index, deg_hist, W_pre1, b_pre1, W_post1, b_post1,
           W_lin1, b_lin1, W_pre2, b_pre2, W_post2, b_post2, W_lin2, b_lin2):
    f32 = jnp.float32
    x = x.astype(f32)
    ei = edge_index.astype(jnp.int32)
    src = ei[0]
    dst = ei[1]
    xp = jnp.zeros((NPAD, D), f32).at[:N].set(x)

    bins = jnp.arange(deg_hist.shape[0], dtype=f32)
    hist = deg_hist.astype(f32)
    avg = jnp.sum(jnp.log(bins + 1.0) * hist) / jnp.sum(hist)
    avg_arr = jnp.full((8, 128), avg, f32)

    cnt, srcl, dstl = _filter(src, dst)

    h = xp
    layers = (
        (W_pre1, b_pre1, W_post1, b_post1, W_lin1, b_lin1, True),
        (W_pre2, b_pre2, W_post2, b_post2, W_lin2, b_lin2, False),
    )
    for Wpre, bpre, Wpost, bpost, Wlin, blin, relu in layers:
        A, B = _pre(h, Wpre[:D].astype(f32), Wpre[D:].astype(f32),
                    bpre.reshape(1, D).astype(f32))
        S1, S2, MN, MX, DEG = _agg(B, cnt, srcl, dstl)
        post = _post_relu if relu else _post_none
        h = post(h, A, S1, S2, MN, MX, DEG, avg_arr,
                 Wpost.reshape(13, D, D).astype(f32),
                 bpost.reshape(1, D).astype(f32), Wlin.astype(f32),
                 blin.reshape(1, D).astype(f32))
    return h[:N].astype(jnp.float64)


# speedup vs baseline: 174.6849x; 1.9181x over previous
import functools

import jax
import jax.numpy as jnp
from jax import lax
from jax.experimental import pallas as pl
from jax.experimental.pallas import tpu as pltpu
from jax.experimental.pallas import tpu_sc as plsc

N = 10000
E = 320000
D = 128
NPAD = 10240
NC = 2
NS = 16
NW = NC * NS
NPT = NPAD // NW
CAP = 11264
K = 64
CH = 8000
BLK = 1024


def _sc_mesh():
    return plsc.VectorSubcoreMesh(
        core_axis_name="c", subcore_axis_name="s", num_cores=NC, num_subcores=NS)


def _filter_body(src_hbm, dst_hbm, cnt_hbm, srcl_hbm, dstl_hbm, deg_hbm,
                 sbuf, dbuf, slist, dlist, degacc, cvec):
    wid = lax.axis_index("s") * NC + lax.axis_index("c")
    lo = wid * NPT
    hi = lo + NPT
    base = wid * NPT

    def fill(i, pos):
        slist[pl.ds(pos, 16)] = jnp.zeros((16,), jnp.int32) + wid
        dlist[pl.ds(pos, 16)] = jnp.zeros((16,), jnp.int32) + NPT
        return pos + 16
    lax.fori_loop(0, CAP // 16, fill, jnp.int32(0))

    def chunk(c, carry):
        off, ebase = carry
        pltpu.sync_copy(src_hbm.at[pl.ds(pl.multiple_of(ebase, 8), CH)], sbuf)
        pltpu.sync_copy(dst_hbm.at[pl.ds(pl.multiple_of(ebase, 8), CH)], dbuf)

        def step(j, carry2):
            off, jpos = carry2
            dv = dbuf[pl.ds(jpos, 16)]
            sv = sbuf[pl.ds(jpos, 16)]
            m = jnp.logical_and(dv >= lo, dv < hi)
            m = jnp.logical_and(m, off <= CAP - 16)
            cs = plsc.cumsum(m.astype(jnp.int32))
            dest = off + cs - 1
            plsc.store_scatter(slist, [dest], sv, mask=m)
            plsc.store_scatter(dlist, [dest], dv - lo, mask=m)
            off = off + cs[15]
            return (off, jpos + 16)
        off, _ = lax.fori_loop(0, CH // 16, step, (off, jnp.int32(0)))
        return (off, ebase + CH)

    off, _ = lax.fori_loop(0, E // CH, chunk, (jnp.int32(0), jnp.int32(0)))
    cvec[...] = jnp.zeros((16,), jnp.int32) + off
    pltpu.sync_copy(cvec, cnt_hbm.at[pl.ds(wid * 16, 16)])
    pltpu.sync_copy(slist, srcl_hbm.at[pl.ds(wid * CAP, CAP)])
    pltpu.sync_copy(dlist, dstl_hbm.at[pl.ds(wid * CAP, CAP)])

    zeros = jnp.zeros((16,), jnp.float32)
    ones = jnp.ones((16,), jnp.float32)

    def dinit(i, pos):
        degacc[pos] = zeros
        return pos + 1
    lax.fori_loop(0, NPT + 1, dinit, jnp.int32(0))

    def dgrp(g, gpos):
        dvec = dlist[pl.ds(gpos, 16)]
        for lane in range(16):
            plsc.addupdate(degacc.at[dvec[lane]], ones)
        return gpos + 16
    lax.fori_loop(0, CAP // 16, dgrp, jnp.int32(0))
    pltpu.sync_copy(degacc.at[pl.ds(jnp.int32(0), NPT)],
                    deg_hbm.at[pl.ds(base, NPT)])


_filter = pl.kernel(
    _filter_body,
    out_type=[
        jax.ShapeDtypeStruct((NW * 16,), jnp.int32),
        jax.ShapeDtypeStruct((NW * CAP,), jnp.int32),
        jax.ShapeDtypeStruct((NW * CAP,), jnp.int32),
        jax.ShapeDtypeStruct((NPAD, 16), jnp.float32),
    ],
    mesh=_sc_mesh(),
    scratch_types=[
        pltpu.VMEM((CH,), jnp.int32),
        pltpu.VMEM((CH,), jnp.int32),
        pltpu.VMEM((CAP,), jnp.int32),
        pltpu.VMEM((CAP,), jnp.int32),
        pltpu.VMEM((NPT + 1, 16), jnp.float32),
        pltpu.VMEM((16,), jnp.int32),
    ],
    compiler_params=pltpu.CompilerParams(needs_layout_passes=False),
    name="pna_sc_filter",
)


def _agg_body(b_hbm, cnt_hbm, srcl_hbm, dstl_hbm,
              s1_hbm, s2_hbm, mn_hbm, mx_hbm,
              slist, dlist, rows0, rows1, acc_a, acc_b, cvec, sem0, sem1):
    wid = lax.axis_index("s") * NC + lax.axis_index("c")
    base = wid * NPT
    lbase = wid * CAP
    pltpu.sync_copy(cnt_hbm.at[pl.ds(wid * 16, 16)], cvec)
    cnt = cvec[pl.ds(0, 16)][0]
    nb = lax.div(cnt + jnp.int32(K - 1), jnp.int32(K))
    nb2 = lax.div(nb + jnp.int32(1), jnp.int32(2))
    capk = jnp.int32(CAP - K)

    pltpu.sync_copy(srcl_hbm.at[pl.ds(pl.multiple_of(lbase, 8), CAP)], slist)
    pltpu.sync_copy(dstl_hbm.at[pl.ds(pl.multiple_of(lbase, 8), CAP)], dlist)

    zeros = jnp.zeros((16,), jnp.float32)
    pinf = jnp.full((16,), jnp.inf, jnp.float32)
    ninf = jnp.full((16,), -jnp.inf, jnp.float32)

    bufs = ((rows0, sem0), (rows1, sem1))

    def start(bi, off):
        rows, sem = bufs[bi]
        pltpu.async_copy(b_hbm.at[slist.at[pl.ds(pl.multiple_of(off, 8), K)]], rows, sem)

    def wait(bi):
        rows, sem = bufs[bi]
        pltpu.make_async_copy(b_hbm.at[slist.at[pl.ds(jnp.int32(0), K)]],
                              rows, sem).wait()

    for phase in (0, 1):
        def init(i, pos):
            for r in range(8):
                sl = pl.ds(r * 16, 16)
                acc_a[pos, sl] = zeros if phase == 0 else pinf
                acc_b[pos, sl] = zeros if phase == 0 else ninf
            return pos + 1
        lax.fori_loop(0, NPT + 1, init, jnp.int32(0))

        def process(bi, off):
            rows, _ = bufs[bi]

            def grp(g, gpos):
                dvec = dlist[pl.ds(gpos, 16)]
                for lane in range(16):
                    dloc = dvec[lane]
                    e = gpos - off + lane
                    for r in range(8):
                        sl = pl.ds(r * 16, 16)
                        v = rows[e, sl]
                        if phase == 0:
                            plsc.addupdate(acc_a.at[dloc, sl], v)
                            plsc.addupdate(acc_b.at[dloc, sl], v * v)
                        else:
                            acc_a[dloc, sl] = jnp.minimum(acc_a[dloc, sl], v)
                            acc_b[dloc, sl] = jnp.maximum(acc_b[dloc, sl], v)
                return gpos + 16
            lax.fori_loop(0, K // 16, grp, off)

        start(0, jnp.int32(0))

        def pair(p, boff):
            poff1 = jnp.minimum(boff + K, capk)
            start(1, poff1)
            wait(0)
            process(0, boff)
            poff2 = jnp.minimum(boff + 2 * K, capk)
            start(0, poff2)
            wait(1)
            process(1, poff1)
            return boff + 2 * K
        lax.fori_loop(0, nb2, pair, jnp.int32(0))
        wait(0)

        npt0 = pl.ds(jnp.int32(0), NPT)
        if phase == 0:
            pltpu.sync_copy(acc_a.at[npt0], s1_hbm.at[pl.ds(base, NPT)])
            pltpu.sync_copy(acc_b.at[npt0], s2_hbm.at[pl.ds(base, NPT)])
        else:
            pltpu.sync_copy(acc_a.at[npt0], mn_hbm.at[pl.ds(base, NPT)])
            pltpu.sync_copy(acc_b.at[npt0], mx_hbm.at[pl.ds(base, NPT)])


_agg = pl.kernel(
    _agg_body,
    out_type=[
        jax.ShapeDtypeStruct((NPAD, D), jnp.float32),
        jax.ShapeDtypeStruct((NPAD, D), jnp.float32),
        jax.ShapeDtypeStruct((NPAD, D), jnp.float32),
        jax.ShapeDtypeStruct((NPAD, D), jnp.float32),
    ],
    mesh=_sc_mesh(),
    scratch_types=[
        pltpu.VMEM((CAP,), jnp.int32),
        pltpu.VMEM((CAP,), jnp.int32),
        pltpu.VMEM((K, D), jnp.float32),
        pltpu.VMEM((K, D), jnp.float32),
        pltpu.VMEM((NPT + 1, D), jnp.float32),
        pltpu.VMEM((NPT + 1, D), jnp.float32),
        pltpu.VMEM((16,), jnp.int32),
        pltpu.SemaphoreType.DMA,
        pltpu.SemaphoreType.DMA,
    ],
    compiler_params=pltpu.CompilerParams(needs_layout_passes=False),
    name="pna_sc_agg",
)


def _pre_body(x_ref, wd_ref, ws_ref, bp_ref, a_ref, b_ref):
    xb = x_ref[...]
    a_ref[...] = (jnp.dot(xb, wd_ref[...], preferred_element_type=jnp.float32, precision=lax.Precision.HIGHEST)
                  + bp_ref[...])
    b_ref[...] = jnp.dot(xb, ws_ref[...], preferred_element_type=jnp.float32, precision=lax.Precision.HIGHEST)


_pre = pl.pallas_call(
    _pre_body,
    grid=(NPAD // BLK,),
    in_specs=[
        pl.BlockSpec((BLK, D), lambda i: (i, jnp.int32(0))),
        pl.BlockSpec((D, D), lambda i: (jnp.int32(0), jnp.int32(0))),
        pl.BlockSpec((D, D), lambda i: (jnp.int32(0), jnp.int32(0))),
        pl.BlockSpec((1, D), lambda i: (jnp.int32(0), jnp.int32(0))),
    ],
    out_specs=[
        pl.BlockSpec((BLK, D), lambda i: (i, jnp.int32(0))),
        pl.BlockSpec((BLK, D), lambda i: (i, jnp.int32(0))),
    ],
    out_shape=[
        jax.ShapeDtypeStruct((NPAD, D), jnp.float32),
        jax.ShapeDtypeStruct((NPAD, D), jnp.float32),
    ],
)


def _post_body(x_ref, a_ref, s1_ref, s2_ref, mn_ref, mx_ref, deg_ref, avg_ref,
               wp_ref, bp_ref, wl_ref, bl_ref, o_ref, *, relu):
    xb = x_ref[...]
    A = a_ref[...]
    S1 = s1_ref[...]
    S2 = s2_ref[...]
    MN = mn_ref[...]
    MX = mx_ref[...]
    deg = deg_ref[...][:, :1]
    degc = jnp.maximum(deg, 1.0)
    has = deg > 0
    logd = jnp.log(degc + 1.0)
    mean = (deg * A + S1) / degc
    meanB = S1 / degc
    varB = S2 / degc - meanB * meanB
    std = jnp.sqrt(jnp.maximum(varB, 0.0) + 1e-5)
    mn = jnp.where(has, A + MN, 0.0)
    mx = jnp.where(has, A + MX, 0.0)
    avg = avg_ref[...][:1, :1]
    sa = logd / avg
    sb = avg / logd
    parts = (xb, mean, mn, mx, std,
             sa * mean, sa * mn, sa * mx, sa * std,
             sb * mean, sb * mn, sb * mx, sb * std)
    y = bp_ref[...] + jnp.zeros((BLK, D), jnp.float32)
    for j in range(13):
        y = y + jnp.dot(parts[j], wp_ref[j], preferred_element_type=jnp.float32, precision=lax.Precision.HIGHEST)
    z = jnp.dot(y, wl_ref[...], preferred_element_type=jnp.float32, precision=lax.Precision.HIGHEST) + bl_ref[...]
    o_ref[...] = jnp.maximum(z, 0.0) if relu else z


def _make_post(relu):
    return pl.pallas_call(
        functools.partial(_post_body, relu=relu),
        grid=(NPAD // BLK,),
        in_specs=[
            pl.BlockSpec((BLK, D), lambda i: (i, jnp.int32(0))),
            pl.BlockSpec((BLK, D), lambda i: (i, jnp.int32(0))),
            pl.BlockSpec((BLK, D), lambda i: (i, jnp.int32(0))),
            pl.BlockSpec((BLK, D), lambda i: (i, jnp.int32(0))),
            pl.BlockSpec((BLK, D), lambda i: (i, jnp.int32(0))),
            pl.BlockSpec((BLK, D), lambda i: (i, jnp.int32(0))),
            pl.BlockSpec((BLK, 16), lambda i: (i, jnp.int32(0))),
            pl.BlockSpec((8, 128), lambda i: (jnp.int32(0), jnp.int32(0))),
            pl.BlockSpec((13, D, D), lambda i: (jnp.int32(0), jnp.int32(0), jnp.int32(0))),
            pl.BlockSpec((1, D), lambda i: (jnp.int32(0), jnp.int32(0))),
            pl.BlockSpec((D, D), lambda i: (jnp.int32(0), jnp.int32(0))),
            pl.BlockSpec((1, D), lambda i: (jnp.int32(0), jnp.int32(0))),
        ],
        out_specs=pl.BlockSpec((BLK, D), lambda i: (i, jnp.int32(0))),
        out_shape=jax.ShapeDtypeStruct((NPAD, D), jnp.float32),
    )


_post_relu = _make_post(True)
_post_none = _make_post(False)


def kernel(x, edge_index, deg_hist, W_pre1, b_pre1, W_post1, b_post1,
           W_lin1, b_lin1, W_pre2, b_pre2, W_post2, b_post2, W_lin2, b_lin2):
    f32 = jnp.float32
    x = x.astype(f32)
    ei = edge_index.astype(jnp.int32)
    src = ei[0]
    dst = ei[1]
    xp = jnp.zeros((NPAD, D), f32).at[:N].set(x)

    bins = jnp.arange(deg_hist.shape[0], dtype=f32)
    hist = deg_hist.astype(f32)
    avg = jnp.sum(jnp.log(bins + 1.0) * hist) / jnp.sum(hist)
    avg_arr = jnp.full((8, 128), avg, f32)

    cnt, srcl, dstl, DEG = _filter(src, dst)

    h = xp
    layers = (
        (W_pre1, b_pre1, W_post1, b_post1, W_lin1, b_lin1, True),
        (W_pre2, b_pre2, W_post2, b_post2, W_lin2, b_lin2, False),
    )
    for Wpre, bpre, Wpost, bpost, Wlin, blin, relu in layers:
        A, B = _pre(h, Wpre[:D].astype(f32), Wpre[D:].astype(f32),
                    bpre.reshape(1, D).astype(f32))
        S1, S2, MN, MX = _agg(B, cnt, srcl, dstl)
        post = _post_relu if relu else _post_none
        h = post(h, A, S1, S2, MN, MX, DEG, avg_arr,
                 Wpost.reshape(13, D, D).astype(f32),
                 bpost.reshape(1, D).astype(f32), Wlin.astype(f32),
                 blin.reshape(1, D).astype(f32))
    return h[:N].astype(jnp.float64)


# speedup vs baseline: 184.9340x; 1.0587x over previous
import functools

import jax
import jax.numpy as jnp
from jax import lax
from jax.experimental import pallas as pl
from jax.experimental.pallas import tpu as pltpu
from jax.experimental.pallas import tpu_sc as plsc

N = 10000
E = 320000
D = 128
NPAD = 10240
NC = 2
NS = 16
NW = NC * NS
NPT = NPAD // NW
CAP = 11264
K = 88
CH = 8000
BLK = 1024

_PERM = [32 * c + (j // 2 if j % 2 == 0 else 16 + j // 2)
         for c in range(4) for j in range(32)]


def _sc_mesh():
    return plsc.VectorSubcoreMesh(
        core_axis_name="c", subcore_axis_name="s", num_cores=NC, num_subcores=NS)


def _filter_body(src_hbm, dst_hbm, cnt_hbm, srcl_hbm, dstl_hbm, deg_hbm,
                 sbuf, dbuf, slist, dlist, degacc, cvec):
    wid = lax.axis_index("s") * NC + lax.axis_index("c")
    lo = wid * NPT
    hi = lo + NPT
    base = wid * NPT

    def fill(i, pos):
        slist[pl.ds(pos, 16)] = jnp.zeros((16,), jnp.int32) + wid
        dlist[pl.ds(pos, 16)] = jnp.zeros((16,), jnp.int32) + NPT
        return pos + 16
    lax.fori_loop(0, CAP // 16, fill, jnp.int32(0))

    def chunk(c, carry):
        off, ebase = carry
        pltpu.sync_copy(src_hbm.at[pl.ds(pl.multiple_of(ebase, 8), CH)], sbuf)
        pltpu.sync_copy(dst_hbm.at[pl.ds(pl.multiple_of(ebase, 8), CH)], dbuf)

        def step(j, carry2):
            off, jpos = carry2
            dv = dbuf[pl.ds(jpos, 16)]
            sv = sbuf[pl.ds(jpos, 16)]
            m = jnp.logical_and(dv >= lo, dv < hi)
            m = jnp.logical_and(m, off <= CAP - 16)
            cs = plsc.cumsum(m.astype(jnp.int32))
            dest = off + cs - 1
            plsc.store_scatter(slist, [dest], sv, mask=m)
            plsc.store_scatter(dlist, [dest], dv - lo, mask=m)
            off = off + cs[15]
            return (off, jpos + 16)
        off, _ = lax.fori_loop(0, CH // 16, step, (off, jnp.int32(0)))
        return (off, ebase + CH)

    off, _ = lax.fori_loop(0, E // CH, chunk, (jnp.int32(0), jnp.int32(0)))
    cvec[...] = jnp.zeros((16,), jnp.int32) + off
    pltpu.sync_copy(cvec, cnt_hbm.at[pl.ds(wid * 16, 16)])
    pltpu.sync_copy(slist, srcl_hbm.at[pl.ds(wid * CAP, CAP)])
    pltpu.sync_copy(dlist, dstl_hbm.at[pl.ds(wid * CAP, CAP)])

    zeros = jnp.zeros((16,), jnp.float32)
    ones = jnp.ones((16,), jnp.float32)

    def dinit(i, pos):
        degacc[pos] = zeros
        return pos + 1
    lax.fori_loop(0, NPT + 1, dinit, jnp.int32(0))

    def dgrp(g, gpos):
        dvec = dlist[pl.ds(gpos, 16)]
        for lane in range(16):
            plsc.addupdate(degacc.at[dvec[lane]], ones)
        return gpos + 16
    lax.fori_loop(0, CAP // 16, dgrp, jnp.int32(0))
    pltpu.sync_copy(degacc.at[pl.ds(jnp.int32(0), NPT)],
                    deg_hbm.at[pl.ds(base, NPT)])


_filter = pl.kernel(
    _filter_body,
    out_type=[
        jax.ShapeDtypeStruct((NW * 16,), jnp.int32),
        jax.ShapeDtypeStruct((NW * CAP,), jnp.int32),
        jax.ShapeDtypeStruct((NW * CAP,), jnp.int32),
        jax.ShapeDtypeStruct((NPAD, 16), jnp.float32),
    ],
    mesh=_sc_mesh(),
    scratch_types=[
        pltpu.VMEM((CH,), jnp.int32),
        pltpu.VMEM((CH,), jnp.int32),
        pltpu.VMEM((CAP,), jnp.int32),
        pltpu.VMEM((CAP,), jnp.int32),
        pltpu.VMEM((NPT + 1, 16), jnp.float32),
        pltpu.VMEM((16,), jnp.int32),
    ],
    compiler_params=pltpu.CompilerParams(needs_layout_passes=False),
    name="pna_sc_filter",
)


def _agg_body(b_hbm, cnt_hbm, srcl_hbm, dstl_hbm,
              s1_hbm, s2_hbm, mn_hbm, mx_hbm,
              slist, dlist, rows0, rows1, acc_a, acc_b, cvec, sem0, sem1):
    wid = lax.axis_index("s") * NC + lax.axis_index("c")
    base = wid * NPT
    lbase = wid * CAP
    pltpu.sync_copy(cnt_hbm.at[pl.ds(wid * 16, 16)], cvec)
    cnt = cvec[pl.ds(0, 16)][0]
    nb = lax.div(cnt + jnp.int32(K - 1), jnp.int32(K))
    nb2 = lax.div(nb + jnp.int32(1), jnp.int32(2))
    capk = jnp.int32(CAP - K)

    pltpu.sync_copy(srcl_hbm.at[pl.ds(pl.multiple_of(lbase, 8), CAP)], slist)
    pltpu.sync_copy(dstl_hbm.at[pl.ds(pl.multiple_of(lbase, 8), CAP)], dlist)

    zeros = jnp.zeros((16,), jnp.float32)
    pinf = jnp.full((16,), jnp.inf, jnp.float32)
    ninf = jnp.full((16,), -jnp.inf, jnp.float32)

    bufs = ((rows0, sem0), (rows1, sem1))

    def start(bi, off):
        rows, sem = bufs[bi]
        pltpu.async_copy(b_hbm.at[slist.at[pl.ds(pl.multiple_of(off, 8), K)]], rows, sem)

    def wait(bi):
        rows, sem = bufs[bi]
        pltpu.make_async_copy(b_hbm.at[slist.at[pl.ds(jnp.int32(0), K)]],
                              rows, sem).wait()

    for phase in (0, 1):
        def init(i, pos):
            for r in range(8):
                sl = pl.ds(r * 16, 16)
                acc_a[pos, sl] = zeros if phase == 0 else pinf
                acc_b[pos, sl] = zeros if phase == 0 else ninf
            return pos + 1
        lax.fori_loop(0, NPT + 1, init, jnp.int32(0))

        def process(bi, off):
            rows, _ = bufs[bi]

            def grp(g, gpos):
                dvec = dlist[pl.ds(gpos, 16)]
                for lane in range(16):
                    dloc = dvec[lane]
                    e = gpos - off + lane
                    for r in range(8):
                        sl = pl.ds(r * 16, 16)
                        v = rows[e, sl]
                        if phase == 0:
                            plsc.addupdate(acc_a.at[dloc, sl], v)
                            plsc.addupdate(acc_b.at[dloc, sl], v * v)
                        else:
                            acc_a[dloc, sl] = jnp.minimum(acc_a[dloc, sl], v)
                            acc_b[dloc, sl] = jnp.maximum(acc_b[dloc, sl], v)
                return gpos + 16
            lax.fori_loop(0, K // 16, grp, off)

        start(0, jnp.int32(0))

        def pair(p, boff):
            poff1 = jnp.minimum(boff + K, capk)
            start(1, poff1)
            wait(0)
            process(0, boff)
            poff2 = jnp.minimum(boff + 2 * K, capk)
            start(0, poff2)
            wait(1)
            process(1, poff1)
            return boff + 2 * K
        lax.fori_loop(0, nb2, pair, jnp.int32(0))
        wait(0)

        npt0 = pl.ds(jnp.int32(0), NPT)
        if phase == 0:
            pltpu.sync_copy(acc_a.at[npt0], s1_hbm.at[pl.ds(base, NPT)])
            pltpu.sync_copy(acc_b.at[npt0], s2_hbm.at[pl.ds(base, NPT)])
        else:
            pltpu.sync_copy(acc_a.at[npt0], mn_hbm.at[pl.ds(base, NPT)])
            pltpu.sync_copy(acc_b.at[npt0], mx_hbm.at[pl.ds(base, NPT)])


_agg = pl.kernel(
    _agg_body,
    out_type=[
        jax.ShapeDtypeStruct((NPAD, D), jnp.float32),
        jax.ShapeDtypeStruct((NPAD, D), jnp.float32),
        jax.ShapeDtypeStruct((NPAD, D), jnp.float32),
        jax.ShapeDtypeStruct((NPAD, D), jnp.float32),
    ],
    mesh=_sc_mesh(),
    scratch_types=[
        pltpu.VMEM((CAP,), jnp.int32),
        pltpu.VMEM((CAP,), jnp.int32),
        pltpu.VMEM((K, D), jnp.float32),
        pltpu.VMEM((K, D), jnp.float32),
        pltpu.VMEM((NPT + 1, D), jnp.float32),
        pltpu.VMEM((NPT + 1, D), jnp.float32),
        pltpu.VMEM((16,), jnp.int32),
        pltpu.SemaphoreType.DMA,
        pltpu.SemaphoreType.DMA,
    ],
    compiler_params=pltpu.CompilerParams(needs_layout_passes=False),
    name="pna_sc_agg",
)


def _pre_body(x_ref, wd_ref, ws_ref, bp_ref, a_ref, b_ref):
    xb = x_ref[...]
    a_ref[...] = (jnp.dot(xb, wd_ref[...], preferred_element_type=jnp.float32, precision=lax.Precision.HIGHEST)
                  + bp_ref[...])
    b_ref[...] = jnp.dot(xb, ws_ref[...], preferred_element_type=jnp.float32, precision=lax.Precision.HIGHEST)


_pre = pl.pallas_call(
    _pre_body,
    grid=(NPAD // BLK,),
    in_specs=[
        pl.BlockSpec((BLK, D), lambda i: (i, jnp.int32(0))),
        pl.BlockSpec((D, D), lambda i: (jnp.int32(0), jnp.int32(0))),
        pl.BlockSpec((D, D), lambda i: (jnp.int32(0), jnp.int32(0))),
        pl.BlockSpec((1, D), lambda i: (jnp.int32(0), jnp.int32(0))),
    ],
    out_specs=[
        pl.BlockSpec((BLK, D), lambda i: (i, jnp.int32(0))),
        pl.BlockSpec((BLK, D), lambda i: (i, jnp.int32(0))),
    ],
    out_shape=[
        jax.ShapeDtypeStruct((NPAD, D), jnp.float32),
        jax.ShapeDtypeStruct((NPAD, D), jnp.float32),
    ],
)


def _post_body(x_ref, a_ref, s1_ref, s2_ref, mn_ref, mx_ref, deg_ref, avg_ref,
               wp_ref, bp_ref, wl_ref, bl_ref, o_ref, *, relu):
    xb = x_ref[...]
    A = a_ref[...]
    S1 = s1_ref[...]
    S2 = s2_ref[...]
    MN = mn_ref[...]
    MX = mx_ref[...]
    deg = deg_ref[...][:, :1]
    degc = jnp.maximum(deg, 1.0)
    has = deg > 0
    logd = jnp.log(degc + 1.0)
    mean = (deg * A + S1) / degc
    meanB = S1 / degc
    varB = S2 / degc - meanB * meanB
    std = jnp.sqrt(jnp.maximum(varB, 0.0) + 1e-5)
    mn = jnp.where(has, A + MN, 0.0)
    mx = jnp.where(has, A + MX, 0.0)
    avg = avg_ref[...][:1, :1]
    sa = logd / avg
    sb = avg / logd
    parts = (xb, mean, mn, mx, std,
             sa * mean, sa * mn, sa * mx, sa * std,
             sb * mean, sb * mn, sb * mx, sb * std)
    y = bp_ref[...] + jnp.zeros((BLK, D), jnp.float32)
    for j in range(13):
        y = y + jnp.dot(parts[j], wp_ref[j], preferred_element_type=jnp.float32, precision=lax.Precision.HIGHEST)
    z = jnp.dot(y, wl_ref[...], preferred_element_type=jnp.float32, precision=lax.Precision.HIGHEST) + bl_ref[...]
    o_ref[...] = jnp.maximum(z, 0.0) if relu else z


def _make_post(relu):
    return pl.pallas_call(
        functools.partial(_post_body, relu=relu),
        grid=(NPAD // BLK,),
        in_specs=[
            pl.BlockSpec((BLK, D), lambda i: (i, jnp.int32(0))),
            pl.BlockSpec((BLK, D), lambda i: (i, jnp.int32(0))),
            pl.BlockSpec((BLK, D), lambda i: (i, jnp.int32(0))),
            pl.BlockSpec((BLK, D), lambda i: (i, jnp.int32(0))),
            pl.BlockSpec((BLK, D), lambda i: (i, jnp.int32(0))),
            pl.BlockSpec((BLK, D), lambda i: (i, jnp.int32(0))),
            pl.BlockSpec((BLK, 16), lambda i: (i, jnp.int32(0))),
            pl.BlockSpec((8, 128), lambda i: (jnp.int32(0), jnp.int32(0))),
            pl.BlockSpec((13, D, D), lambda i: (jnp.int32(0), jnp.int32(0), jnp.int32(0))),
            pl.BlockSpec((1, D), lambda i: (jnp.int32(0), jnp.int32(0))),
            pl.BlockSpec((D, D), lambda i: (jnp.int32(0), jnp.int32(0))),
            pl.BlockSpec((1, D), lambda i: (jnp.int32(0), jnp.int32(0))),
        ],
        out_specs=pl.BlockSpec((BLK, D), lambda i: (i, jnp.int32(0))),
        out_shape=jax.ShapeDtypeStruct((NPAD, D), jnp.float32),
    )


_post_relu = _make_post(True)
_post_none = _make_post(False)


def kernel(x, edge_index, deg_hist, W_pre1, b_pre1, W_post1, b_post1,
           W_lin1, b_lin1, W_pre2, b_pre2, W_post2, b_post2, W_lin2, b_lin2):
    f32 = jnp.float32
    x = x.astype(f32)
    ei = edge_index.astype(jnp.int32)
    src = ei[0]
    dst = ei[1]
    xp = jnp.zeros((NPAD, D), f32).at[:N].set(x)

    bins = jnp.arange(deg_hist.shape[0], dtype=f32)
    hist = deg_hist.astype(f32)
    avg = jnp.sum(jnp.log(bins + 1.0) * hist) / jnp.sum(hist)
    avg_arr = jnp.full((8, 128), avg, f32)

    cnt, srcl, dstl, DEG = _filter(src, dst)

    h = xp
    layers = (
        (W_pre1, b_pre1, W_post1, b_post1, W_lin1, b_lin1, True),
        (W_pre2, b_pre2, W_post2, b_post2, W_lin2, b_lin2, False),
    )
    for Wpre, bpre, Wpost, bpost, Wlin, blin, relu in layers:
        A, B = _pre(h, Wpre[:D].astype(f32), Wpre[D:].astype(f32),
                    bpre.reshape(1, D).astype(f32))
        S1, S2, MN, MX = _agg(B, cnt, srcl, dstl)
        post = _post_relu if relu else _post_none
        h = post(h, A, S1, S2, MN, MX, DEG, avg_arr,
                 Wpost.reshape(13, D, D).astype(f32),
                 bpost.reshape(1, D).astype(f32), Wlin.astype(f32),
                 blin.reshape(1, D).astype(f32))
    return h[:N].astype(jnp.float64)
